# baseline (device time: 26881 ns/iter reference)
import jax
import jax.numpy as jnp
from jax import lax
from jax.experimental import pallas as pl
from jax.experimental.pallas import tpu as pltpu

B = 4
S_HALF = 256
H = 8
D = 64
K = H * D
N = 1024
NH = 512


def kernel(O, Wo):
    ot = jnp.transpose(O, (0, 2, 3, 1))

    def body(ot_hbm, w_hbm, out_hbm, ot_vmem, w_vmem, send_y, recv_y,
             send_x, recv_x, res, resx, o_sems, w_sem, ysend_sems,
             yrecv_sems, xsend_sems, xrecv_sems, out_sems):
        my_x = lax.axis_index("x")
        my_y = lax.axis_index("y")
        peer_y = 1 - my_y
        peer_x = 1 - my_x
        col0 = my_x * NH

        w_dma = pltpu.make_async_copy(
            w_hbm.at[:, pl.ds(col0, NH)], w_vmem, w_sem
        )
        w_dma.start()
        o_dmas = []
        for c in range(B):
            d = pltpu.make_async_copy(ot_hbm.at[c], ot_vmem.at[c],
                                      o_sems.at[c])
            d.start()
            o_dmas.append(d)

        barrier = pltpu.get_barrier_semaphore()
        for dev in ((my_x, peer_y), (peer_x, my_y)):
            pl.semaphore_signal(
                barrier, inc=1,
                device_id=dev, device_id_type=pl.DeviceIdType.MESH,
            )
        pl.semaphore_wait(barrier, 2)

        w_dma.wait()
        w_my = w_vmem[:, :].astype(jnp.bfloat16)

        def chunk_matmul(c, s0):
            lhsT = ot_vmem[c, :, :, pl.ds(s0, S_HALF)].reshape(K, S_HALF)
            return lax.dot_general(
                lhsT.astype(jnp.bfloat16), w_my,
                (((0,), (0,)), ((), ())),
                preferred_element_type=jnp.float32,
            )

        y_rdmas = []
        for c in range(B):
            o_dmas[c].wait()
            send_y[c, :, :] = chunk_matmul(c, peer_y * S_HALF).astype(
                jnp.bfloat16
            )
            r = pltpu.make_async_remote_copy(
                src_ref=send_y.at[c],
                dst_ref=recv_y.at[c],
                send_sem=ysend_sems.at[c],
                recv_sem=yrecv_sems.at[c],
                device_id=(my_x, peer_y),
                device_id_type=pl.DeviceIdType.MESH,
            )
            r.start()
            y_rdmas.append(r)

        x_rdmas = []
        out_dmas = []
        for c in range(B):
            res[c, :, :] = chunk_matmul(c, my_y * S_HALF)
            y_rdmas[c].wait_recv()
            res[c, :, :] = res[c, :, :] + recv_y[c, :, :].astype(jnp.float32)
            send_x[c, :, :] = res[c, :, :].astype(jnp.bfloat16)
            rx = pltpu.make_async_remote_copy(
                src_ref=send_x.at[c],
                dst_ref=recv_x.at[c],
                send_sem=xsend_sems.at[c],
                recv_sem=xrecv_sems.at[c],
                device_id=(peer_x, my_y),
                device_id_type=pl.DeviceIdType.MESH,
            )
            rx.start()
            x_rdmas.append(rx)
            od = pltpu.make_async_copy(
                res.at[c], out_hbm.at[c, :, pl.ds(col0, NH)],
                out_sems.at[c, 0],
            )
            od.start()
            out_dmas.append(od)

        for c in range(B):
            x_rdmas[c].wait_recv()
            resx[c, :, :] = recv_x[c, :, :].astype(jnp.float32)
            od = pltpu.make_async_copy(
                resx.at[c], out_hbm.at[c, :, pl.ds(peer_x * NH, NH)],
                out_sems.at[c, 1],
            )
            od.start()
            out_dmas.append(od)

        for od in out_dmas:
            od.wait()
        for r in y_rdmas:
            r.wait_send()
        for r in x_rdmas:
            r.wait_send()

    return pl.pallas_call(
        body,
        out_shape=jax.ShapeDtypeStruct((B, S_HALF, N), jnp.float32),
        in_specs=[
            pl.BlockSpec(memory_space=pltpu.MemorySpace.HBM),
            pl.BlockSpec(memory_space=pltpu.MemorySpace.HBM),
        ],
        out_specs=pl.BlockSpec(memory_space=pltpu.MemorySpace.HBM),
        scratch_shapes=[
            pltpu.VMEM((B, H, D, 2 * S_HALF), jnp.float32),
            pltpu.VMEM((K, NH), jnp.float32),
            pltpu.VMEM((B, S_HALF, NH), jnp.bfloat16),
            pltpu.VMEM((B, S_HALF, NH), jnp.bfloat16),
            pltpu.VMEM((B, S_HALF, NH), jnp.bfloat16),
            pltpu.VMEM((B, S_HALF, NH), jnp.bfloat16),
            pltpu.VMEM((B, S_HALF, NH), jnp.float32),
            pltpu.VMEM((B, S_HALF, NH), jnp.float32),
            pltpu.SemaphoreType.DMA((B,)),
            pltpu.SemaphoreType.DMA,
            pltpu.SemaphoreType.DMA((B,)),
            pltpu.SemaphoreType.DMA((B,)),
            pltpu.SemaphoreType.DMA((B,)),
            pltpu.SemaphoreType.DMA((B,)),
            pltpu.SemaphoreType.DMA((B, 2)),
        ],
        compiler_params=pltpu.CompilerParams(collective_id=0),
    )(ot, Wo)


# device time: 25142 ns/iter; 1.0692x vs baseline; 1.0692x over previous
import jax
import jax.numpy as jnp
from jax import lax
from jax.experimental import pallas as pl
from jax.experimental.pallas import tpu as pltpu

B = 4
S_HALF = 256
H = 8
D = 64
K = H * D
N = 1024
NH = 512


def kernel(O, Wo):
    ot = jnp.transpose(O, (0, 2, 3, 1))
    ot = pltpu.with_memory_space_constraint(ot, pltpu.MemorySpace.HBM)
    Wo = pltpu.with_memory_space_constraint(Wo, pltpu.MemorySpace.HBM)

    def body(ot_hbm, w_hbm, out_hbm, ot_vmem, w_vmem, send_y, recv_y,
             send_x, recv_x, res, resx, o_sems, w_sem, ysend_sems,
             yrecv_sems, xsend_sems, xrecv_sems, out_sems):
        my_x = lax.axis_index("x")
        my_y = lax.axis_index("y")
        peer_y = 1 - my_y
        peer_x = 1 - my_x
        col0 = my_x * NH

        w_dma = pltpu.make_async_copy(
            w_hbm.at[:, pl.ds(col0, NH)], w_vmem, w_sem
        )
        w_dma.start()
        o_dmas = []
        for c in range(B):
            d = pltpu.make_async_copy(ot_hbm.at[c], ot_vmem.at[c],
                                      o_sems.at[c])
            d.start()
            o_dmas.append(d)

        barrier = pltpu.get_barrier_semaphore()
        for dev in ((my_x, peer_y), (peer_x, my_y)):
            pl.semaphore_signal(
                barrier, inc=1,
                device_id=dev, device_id_type=pl.DeviceIdType.MESH,
            )
        pl.semaphore_wait(barrier, 2)

        w_dma.wait()
        w_my = w_vmem[:, :].astype(jnp.bfloat16)

        def chunk_matmul(c, s0):
            lhsT = ot_vmem[c, :, :, pl.ds(s0, S_HALF)].reshape(K, S_HALF)
            return lax.dot_general(
                lhsT.astype(jnp.bfloat16), w_my,
                (((0,), (0,)), ((), ())),
                preferred_element_type=jnp.float32,
            )

        y_rdmas = []
        for c in range(B):
            o_dmas[c].wait()
            send_y[c, :, :] = chunk_matmul(c, peer_y * S_HALF).astype(
                jnp.bfloat16
            )
            r = pltpu.make_async_remote_copy(
                src_ref=send_y.at[c],
                dst_ref=recv_y.at[c],
                send_sem=ysend_sems.at[c],
                recv_sem=yrecv_sems.at[c],
                device_id=(my_x, peer_y),
                device_id_type=pl.DeviceIdType.MESH,
            )
            r.start()
            y_rdmas.append(r)

        x_rdmas = []
        out_dmas = []
        for c in range(B):
            res[c, :, :] = chunk_matmul(c, my_y * S_HALF)
            y_rdmas[c].wait_recv()
            res[c, :, :] = res[c, :, :] + recv_y[c, :, :].astype(jnp.float32)
            send_x[c, :, :] = res[c, :, :].astype(jnp.bfloat16)
            rx = pltpu.make_async_remote_copy(
                src_ref=send_x.at[c],
                dst_ref=recv_x.at[c],
                send_sem=xsend_sems.at[c],
                recv_sem=xrecv_sems.at[c],
                device_id=(peer_x, my_y),
                device_id_type=pl.DeviceIdType.MESH,
            )
            rx.start()
            x_rdmas.append(rx)
            od = pltpu.make_async_copy(
                res.at[c], out_hbm.at[c, :, pl.ds(col0, NH)],
                out_sems.at[c, 0],
            )
            od.start()
            out_dmas.append(od)

        for c in range(B):
            x_rdmas[c].wait_recv()
            resx[c, :, :] = recv_x[c, :, :].astype(jnp.float32)
            od = pltpu.make_async_copy(
                resx.at[c], out_hbm.at[c, :, pl.ds(peer_x * NH, NH)],
                out_sems.at[c, 1],
            )
            od.start()
            out_dmas.append(od)

        for od in out_dmas:
            od.wait()
        for r in y_rdmas:
            r.wait_send()
        for r in x_rdmas:
            r.wait_send()

    return pl.pallas_call(
        body,
        out_shape=jax.ShapeDtypeStruct((B, S_HALF, N), jnp.float32),
        in_specs=[
            pl.BlockSpec(memory_space=pltpu.MemorySpace.HBM),
            pl.BlockSpec(memory_space=pltpu.MemorySpace.HBM),
        ],
        out_specs=pl.BlockSpec(memory_space=pltpu.MemorySpace.HBM),
        scratch_shapes=[
            pltpu.VMEM((B, H, D, 2 * S_HALF), jnp.float32),
            pltpu.VMEM((K, NH), jnp.float32),
            pltpu.VMEM((B, S_HALF, NH), jnp.bfloat16),
            pltpu.VMEM((B, S_HALF, NH), jnp.bfloat16),
            pltpu.VMEM((B, S_HALF, NH), jnp.bfloat16),
            pltpu.VMEM((B, S_HALF, NH), jnp.bfloat16),
            pltpu.VMEM((B, S_HALF, NH), jnp.float32),
            pltpu.VMEM((B, S_HALF, NH), jnp.float32),
            pltpu.SemaphoreType.DMA((B,)),
            pltpu.SemaphoreType.DMA,
            pltpu.SemaphoreType.DMA((B,)),
            pltpu.SemaphoreType.DMA((B,)),
            pltpu.SemaphoreType.DMA((B,)),
            pltpu.SemaphoreType.DMA((B,)),
            pltpu.SemaphoreType.DMA((B, 2)),
        ],
        compiler_params=pltpu.CompilerParams(collective_id=0),
    )(ot, Wo)


# device time: 23819 ns/iter; 1.1286x vs baseline; 1.0555x over previous
import jax
import jax.numpy as jnp
from jax import lax
from jax.experimental import pallas as pl
from jax.experimental.pallas import tpu as pltpu

B = 4
S_HALF = 256
H = 8
D = 64
K = H * D
N = 1024
NH = 512
CH = 8
R = B * S_HALF // CH


def kernel(O, Wo):
    ot = jnp.transpose(O, (0, 2, 3, 1))
    ot = pltpu.with_memory_space_constraint(ot, pltpu.MemorySpace.HBM)
    Wo = pltpu.with_memory_space_constraint(Wo, pltpu.MemorySpace.HBM)

    def body(ot_hbm, w_hbm, out_hbm, ot_vmem, w_vmem, send_y, recv_y,
             send_x, recv_x, res, resx, o_sems, w_sem, ysend_sems,
             yrecv_sems, xsend_sems, xrecv_sems, out_sems):
        my_x = lax.axis_index("x")
        my_y = lax.axis_index("y")
        peer_y = 1 - my_y
        peer_x = 1 - my_x
        col0 = my_x * NH

        w_dma = pltpu.make_async_copy(
            w_hbm.at[:, pl.ds(col0, NH)], w_vmem, w_sem
        )
        w_dma.start()
        o_dmas = []
        for c in range(B):
            d = pltpu.make_async_copy(ot_hbm.at[c], ot_vmem.at[c],
                                      o_sems.at[c])
            d.start()
            o_dmas.append(d)

        barrier = pltpu.get_barrier_semaphore()
        for dev in ((my_x, peer_y), (peer_x, my_y)):
            pl.semaphore_signal(
                barrier, inc=1,
                device_id=dev, device_id_type=pl.DeviceIdType.MESH,
            )
        pl.semaphore_wait(barrier, 2)

        w_dma.wait()
        w_my = w_vmem[:, :].astype(jnp.bfloat16)

        def chunk_matmul(q, s0):
            lhsT = ot_vmem[
                q // 2, :, :, pl.ds(s0 + (q % 2) * R, R)
            ].reshape(K, R)
            return lax.dot_general(
                lhsT.astype(jnp.bfloat16), w_my,
                (((0,), (0,)), ((), ())),
                preferred_element_type=jnp.float32,
            )

        y_rdmas = []
        for q in range(CH):
            if q % 2 == 0:
                o_dmas[q // 2].wait()
            send_y[q, :, :] = chunk_matmul(q, peer_y * S_HALF).astype(
                jnp.bfloat16
            )
            r = pltpu.make_async_remote_copy(
                src_ref=send_y.at[q],
                dst_ref=recv_y.at[q],
                send_sem=ysend_sems.at[q],
                recv_sem=yrecv_sems.at[q],
                device_id=(my_x, peer_y),
                device_id_type=pl.DeviceIdType.MESH,
            )
            r.start()
            y_rdmas.append(r)

        x_rdmas = []
        out_dmas = []
        for q in range(CH):
            res[q, :, :] = chunk_matmul(q, my_y * S_HALF)
            y_rdmas[q].wait_recv()
            res[q, :, :] = res[q, :, :] + recv_y[q, :, :].astype(jnp.float32)
            send_x[q, :, :] = res[q, :, :].astype(jnp.bfloat16)
            rx = pltpu.make_async_remote_copy(
                src_ref=send_x.at[q],
                dst_ref=recv_x.at[q],
                send_sem=xsend_sems.at[q],
                recv_sem=xrecv_sems.at[q],
                device_id=(peer_x, my_y),
                device_id_type=pl.DeviceIdType.MESH,
            )
            rx.start()
            x_rdmas.append(rx)
            od = pltpu.make_async_copy(
                res.at[q],
                out_hbm.at[q // 2, pl.ds((q % 2) * R, R), pl.ds(col0, NH)],
                out_sems.at[q, 0],
            )
            od.start()
            out_dmas.append(od)

        for q in range(CH):
            x_rdmas[q].wait_recv()
            resx[q, :, :] = recv_x[q, :, :].astype(jnp.float32)
            od = pltpu.make_async_copy(
                resx.at[q],
                out_hbm.at[
                    q // 2, pl.ds((q % 2) * R, R), pl.ds(peer_x * NH, NH)
                ],
                out_sems.at[q, 1],
            )
            od.start()
            out_dmas.append(od)

        for od in out_dmas:
            od.wait()
        for r in y_rdmas:
            r.wait_send()
        for r in x_rdmas:
            r.wait_send()

    return pl.pallas_call(
        body,
        out_shape=jax.ShapeDtypeStruct((B, S_HALF, N), jnp.float32),
        in_specs=[
            pl.BlockSpec(memory_space=pltpu.MemorySpace.HBM),
            pl.BlockSpec(memory_space=pltpu.MemorySpace.HBM),
        ],
        out_specs=pl.BlockSpec(memory_space=pltpu.MemorySpace.HBM),
        scratch_shapes=[
            pltpu.VMEM((B, H, D, 2 * S_HALF), jnp.float32),
            pltpu.VMEM((K, NH), jnp.float32),
            pltpu.VMEM((CH, R, NH), jnp.bfloat16),
            pltpu.VMEM((CH, R, NH), jnp.bfloat16),
            pltpu.VMEM((CH, R, NH), jnp.bfloat16),
            pltpu.VMEM((CH, R, NH), jnp.bfloat16),
            pltpu.VMEM((CH, R, NH), jnp.float32),
            pltpu.VMEM((CH, R, NH), jnp.float32),
            pltpu.SemaphoreType.DMA((B,)),
            pltpu.SemaphoreType.DMA,
            pltpu.SemaphoreType.DMA((CH,)),
            pltpu.SemaphoreType.DMA((CH,)),
            pltpu.SemaphoreType.DMA((CH,)),
            pltpu.SemaphoreType.DMA((CH,)),
            pltpu.SemaphoreType.DMA((CH, 2)),
        ],
        compiler_params=pltpu.CompilerParams(collective_id=0),
    )(ot, Wo)
